# Initial kernel scaffold; baseline (speedup 1.0000x reference)
import functools

import jax
import jax.numpy as jnp
from jax import lax
from jax.experimental import pallas as pl
from jax.experimental.pallas import tpu as pltpu
from jax.experimental.pallas import tpu_sc as plsc

N = 10000
E = 320000
D = 128
H = 128
G = 128

NC = 2
NS = 16
NW = NC * NS
EPW = E // NW
K = 80
NCHUNK = EPW // K
EC = E // K
RPT = N // NS

BN = 1000
NB = N // BN

_mesh = plsc.VectorSubcoreMesh(core_axis_name="c", subcore_axis_name="s")


@functools.partial(
    pl.kernel,
    out_type=jax.ShapeDtypeStruct((NC * N,), jnp.float32),
    mesh=_mesh,
    scratch_types=[
        pltpu.VMEM((NCHUNK, K), jnp.int32),
        pltpu.VMEM((K,), jnp.float32),
        pltpu.VMEM((1000,), jnp.float32),
        pltpu.VMEM_SHARED((N,), jnp.float32),
        pltpu.SemaphoreType.DMA,
    ],
)
def _sc_deg(dst_hbm, out_hbm, dstb, ones_v, zb, deg_sh, sem):
    c = lax.axis_index("c")
    s = lax.axis_index("s")
    w = s * NC + c

    one16 = jnp.ones((16,), jnp.float32)
    zero16 = jnp.zeros((16,), jnp.float32)
    for j in range(K // 16):
        ones_v[pl.ds(j * 16, 16)] = one16

    @pl.loop(0, 1000 // 16)
    def _(i):
        zb[pl.ds(i * 16, 16)] = zero16

    @pl.when(s < 10)
    def _():
        pltpu.sync_copy(zb, deg_sh.at[pl.ds(s * 1000, 1000)])

    pltpu.sync_copy(dst_hbm.at[pl.ds(w * NCHUNK, NCHUNK)], dstb)
    plsc.subcore_barrier()

    AHEAD = 4

    @pl.loop(0, NCHUNK)
    def _(j):
        pltpu.async_copy(ones_v, deg_sh.at[dstb.at[j]], sem, add=True)

        @pl.when(j >= AHEAD)
        def _():
            pltpu.make_async_copy(ones_v, deg_sh.at[dstb.at[0]], sem).wait()

    for _ in range(AHEAD):
        pltpu.make_async_copy(ones_v, deg_sh.at[dstb.at[0]], sem).wait()

    plsc.subcore_barrier()

    @pl.when(s < 10)
    def _():
        pltpu.sync_copy(deg_sh.at[pl.ds(s * 1000, 1000)],
                        out_hbm.at[pl.ds(c * N + s * 1000, 1000)])


@functools.partial(
    pl.kernel,
    out_type=jax.ShapeDtypeStruct((NC * N, H), jnp.float32),
    mesh=_mesh,
    scratch_types=[
        pltpu.VMEM((NCHUNK, K), jnp.int32),
        pltpu.VMEM((NCHUNK, K), jnp.int32),
        pltpu.VMEM((K, H), jnp.float32),
        pltpu.VMEM((K, H), jnp.float32),
        pltpu.VMEM((RPT // 5, H), jnp.float32),
        pltpu.VMEM_SHARED((N, H), jnp.float32),
        pltpu.SemaphoreType.DMA,
        pltpu.SemaphoreType.DMA,
    ],
)
def _sc_agg(g_hbm, src_hbm, dst_hbm, out_hbm,
            srcb, dstb, rows0, rows1, zb, acc_sh, sem0, sem1):
    c = lax.axis_index("c")
    s = lax.axis_index("s")
    w = s * NC + c

    zero16 = jnp.zeros((16,), jnp.float32)

    @pl.loop(0, RPT // 5)
    def _(r):
        for j in range(H // 16):
            zb[r, pl.ds(j * 16, 16)] = zero16

    for t in range(5):
        pltpu.sync_copy(zb, acc_sh.at[pl.ds(s * RPT + t * (RPT // 5), RPT // 5)])

    pltpu.sync_copy(src_hbm.at[pl.ds(w * NCHUNK, NCHUNK)], srcb)
    pltpu.sync_copy(dst_hbm.at[pl.ds(w * NCHUNK, NCHUNK)], dstb)
    plsc.subcore_barrier()

    def start_gather(j, buf, sem):
        pltpu.async_copy(g_hbm.at[srcb.at[j]], buf, sem)

    def wait_gather(j, buf, sem):
        pltpu.make_async_copy(g_hbm.at[srcb.at[j]], buf, sem).wait()

    def scatter(j, buf):
        pltpu.sync_copy(buf, acc_sh.at[dstb.at[j]], add=True)

    start_gather(0, rows0, sem0)
    start_gather(1, rows1, sem1)

    @pl.loop(0, (NCHUNK - 1) // 2)
    def _(t):
        j0 = 2 * t
        wait_gather(j0, rows0, sem0)
        scatter(j0, rows0)

        @pl.when(j0 + 2 < NCHUNK)
        def _():
            start_gather(j0 + 2, rows0, sem0)

        wait_gather(j0 + 1, rows1, sem1)
        scatter(j0 + 1, rows1)

        @pl.when(j0 + 3 < NCHUNK)
        def _():
            start_gather(j0 + 3, rows1, sem1)

    wait_gather(NCHUNK - 1, rows0, sem0)
    scatter(NCHUNK - 1, rows0)

    plsc.subcore_barrier()
    pltpu.sync_copy(acc_sh.at[pl.ds(s * RPT, RPT)],
                    out_hbm.at[pl.ds(c * N + s * RPT, RPT)])


def _prep_body(x_ref, w_ref, dinv_ref, g_ref):
    u = jnp.dot(x_ref[...], w_ref[...], preferred_element_type=jnp.float32)
    g_ref[...] = u * dinv_ref[...]


def _tc_prep(x, W1, dinv):
    return pl.pallas_call(
        _prep_body,
        grid=(NB,),
        in_specs=[
            pl.BlockSpec((BN, D), lambda i: (i, 0)),
            pl.BlockSpec((D, H), lambda i: (0, 0)),
            pl.BlockSpec((BN, 1), lambda i: (i, 0)),
        ],
        out_specs=pl.BlockSpec((BN, H), lambda i: (i, 0)),
        out_shape=jax.ShapeDtypeStruct((N, H), jnp.float32),
    )(x, W1, dinv)


def _mid_body(p0_ref, p1_ref, g_ref, dinv_ref, b_ref, w_ref, o_ref):
    z = dinv_ref[...] * (p0_ref[...] + p1_ref[...] + g_ref[...]) + b_ref[...]
    z = jnp.maximum(z, 0.0)
    u = jnp.dot(z, w_ref[...], preferred_element_type=jnp.float32)
    o_ref[...] = u * dinv_ref[...]


def _tc_mid(agg, g, dinv, b, W):
    return pl.pallas_call(
        _mid_body,
        grid=(NB,),
        in_specs=[
            pl.BlockSpec((BN, H), lambda i: (i, 0)),
            pl.BlockSpec((BN, H), lambda i: (i + NB, 0)),
            pl.BlockSpec((BN, H), lambda i: (i, 0)),
            pl.BlockSpec((BN, 1), lambda i: (i, 0)),
            pl.BlockSpec((1, H), lambda i: (0, 0)),
            pl.BlockSpec((H, H), lambda i: (0, 0)),
        ],
        out_specs=pl.BlockSpec((BN, H), lambda i: (i, 0)),
        out_shape=jax.ShapeDtypeStruct((N, H), jnp.float32),
    )(agg, agg, g, dinv, b, W)


def _final_body(p0_ref, p1_ref, g_ref, dinv_ref, b_ref, batch_ref, wo_ref,
                bo_ref, o_ref, acc, cnt):
    i = pl.program_id(0)

    @pl.when(i == 0)
    def _():
        acc[...] = jnp.zeros_like(acc)
        cnt[...] = jnp.zeros_like(cnt)

    z = dinv_ref[...] * (p0_ref[...] + p1_ref[...] + g_ref[...]) + b_ref[...]
    z = jnp.maximum(z, 0.0)
    q = jnp.dot(z, wo_ref[...], preferred_element_type=jnp.float32)
    gids = lax.broadcasted_iota(jnp.int32, (1, G), 1)
    m = (batch_ref[...] == gids).astype(jnp.float32)
    dn = (((0,), (0,)), ((), ()))
    acc[...] += lax.dot_general(m, q, dn, preferred_element_type=jnp.float32)
    cnt[...] += lax.dot_general(m, jnp.ones((BN, 1), jnp.float32), dn,
                                preferred_element_type=jnp.float32)

    @pl.when(i == NB - 1)
    def _():
        o_ref[...] = acc[...] / jnp.maximum(cnt[...], 1.0) + bo_ref[...]


def _tc_final(agg, g, dinv, b, batch2d, Wo, bo2d):
    return pl.pallas_call(
        _final_body,
        grid=(NB,),
        in_specs=[
            pl.BlockSpec((BN, H), lambda i: (i, 0)),
            pl.BlockSpec((BN, H), lambda i: (i + NB, 0)),
            pl.BlockSpec((BN, H), lambda i: (i, 0)),
            pl.BlockSpec((BN, 1), lambda i: (i, 0)),
            pl.BlockSpec((1, H), lambda i: (0, 0)),
            pl.BlockSpec((BN, 1), lambda i: (i, 0)),
            pl.BlockSpec((H, 1), lambda i: (0, 0)),
            pl.BlockSpec((1, 1), lambda i: (0, 0)),
        ],
        out_specs=pl.BlockSpec((G, 1), lambda i: (0, 0)),
        out_shape=jax.ShapeDtypeStruct((G, 1), jnp.float32),
        scratch_shapes=[
            pltpu.VMEM((G, 1), jnp.float32),
            pltpu.VMEM((G, 1), jnp.float32),
        ],
    )(agg, agg, g, dinv, b, batch2d, Wo, bo2d)


def kernel(x, edge_index, batch, W1, b1, W2, b2, Wo, bo):
    src2 = edge_index[0].reshape(EC, K)
    dst2 = edge_index[1].reshape(EC, K)

    degp = _sc_deg(dst2)
    dinv = lax.rsqrt(degp[:N] + degp[N:] + 1.0).reshape(N, 1)

    g1 = _tc_prep(x, W1, dinv)
    a1 = _sc_agg(g1, src2, dst2)
    g2 = _tc_mid(a1, g1, dinv, b1.reshape(1, H), W2)
    a2 = _sc_agg(g2, src2, dst2)
    out = _tc_final(a2, g2, dinv, b2.reshape(1, H), batch.reshape(N, 1),
                    Wo, bo.reshape(1, 1))
    return out.reshape(-1)


# speedup vs baseline: 22.9255x; 22.9255x over previous
import functools

import jax
import jax.numpy as jnp
from jax import lax
from jax.experimental import pallas as pl
from jax.experimental.pallas import tpu as pltpu
from jax.experimental.pallas import tpu_sc as plsc

N = 10000
E = 320000
D = 128
H = 128
G = 128

NC = 2
NS = 16
NW = NC * NS
EPW = E // NW
K = 80
NCHUNK = EPW // K
NP_ = 10240
RPT = NP_ // NS

BN = 1000
NB = N // BN

def _mesh():
    return plsc.VectorSubcoreMesh(core_axis_name="c", subcore_axis_name="s",
                                  num_cores=NC, num_subcores=NS)


def _sc_deg_body(dst_hbm, out_hbm, dstb, ones_v, zb, deg_sh, sem):
    c = lax.axis_index("c")
    s = lax.axis_index("s")
    w = s * NC + c

    one16 = jnp.ones((16,), jnp.float32)
    zero16 = jnp.zeros((16,), jnp.float32)
    for j in range(K // 16):
        ones_v[pl.ds(j * 16, 16)] = one16

    @pl.loop(0, 1000 // 16)
    def _(i):
        zb[pl.ds(i * 16, 16)] = zero16

    @pl.when(s < 10)
    def _():
        pltpu.sync_copy(zb, deg_sh.at[pl.ds(pl.multiple_of(s * 1000, 8), 1000)])

    pltpu.sync_copy(dst_hbm.at[w], dstb)
    plsc.subcore_barrier()

    AHEAD = 4

    @pl.loop(0, NCHUNK)
    def _(j):
        pltpu.async_copy(ones_v, deg_sh.at[dstb.at[j]], sem, add=True)

        @pl.when(j >= AHEAD)
        def _():
            pltpu.make_async_copy(ones_v, deg_sh.at[dstb.at[0]], sem).wait()

    for _ in range(AHEAD):
        pltpu.make_async_copy(ones_v, deg_sh.at[dstb.at[0]], sem).wait()

    plsc.subcore_barrier()

    @pl.when(s < 10)
    def _():
        pltpu.sync_copy(deg_sh.at[pl.ds(pl.multiple_of(s * 1000, 8), 1000)],
                        zb)
        pltpu.sync_copy(zb,
                        out_hbm.at[pl.ds(pl.multiple_of(c * N + s * 1000, 8),
                                         1000)])


@functools.cache
def _sc_deg_kernel():
    return pl.kernel(
        _sc_deg_body,
        out_type=jax.ShapeDtypeStruct((NC * N,), jnp.float32),
        mesh=_mesh(),
        compiler_params=pltpu.CompilerParams(use_tc_tiling_on_sc=False),
        scratch_types=[
            pltpu.VMEM((NCHUNK, K), jnp.int32),
            pltpu.VMEM((K,), jnp.float32),
            pltpu.VMEM((1000,), jnp.float32),
            pltpu.VMEM_SHARED((N,), jnp.float32),
            pltpu.SemaphoreType.DMA,
        ],
    )


def _sc_deg(dst3):
    return _sc_deg_kernel()(dst3)


HH = H // 2


def _sc_agg_body(ga_hbm, gb_hbm, src_hbm, dst_hbm, outa_hbm, outb_hbm,
                 srcb, dstb, rows0, rows1, zb, acc_sh, sem0, sem1):
    c = lax.axis_index("c")
    s = lax.axis_index("s")
    w = s * NC + c

    zero16 = jnp.zeros((16,), jnp.float32)
    CH = RPT // 5

    @pl.loop(0, CH)
    def _(r):
        for j in range(HH // 16):
            zb[r, pl.ds(j * 16, 16)] = zero16

    pltpu.sync_copy(src_hbm.at[w], srcb)
    pltpu.sync_copy(dst_hbm.at[w], dstb)

    for g_hbm, out_hbm in ((ga_hbm, outa_hbm), (gb_hbm, outb_hbm)):
        for t in range(5):
            pltpu.sync_copy(
                zb,
                acc_sh.at[pl.ds(pl.multiple_of(s * RPT + t * CH, 8), CH)])
        plsc.subcore_barrier()

        def start_gather(j, buf, sem):
            pltpu.async_copy(g_hbm.at[srcb.at[j]], buf, sem)

        def wait_gather(j, buf, sem):
            pltpu.make_async_copy(g_hbm.at[srcb.at[j]], buf, sem).wait()

        def scatter(j, buf):
            pltpu.sync_copy(buf, acc_sh.at[dstb.at[j]], add=True)

        start_gather(0, rows0, sem0)
        start_gather(1, rows1, sem1)

        @pl.loop(0, (NCHUNK - 1) // 2)
        def _(t):
            j0 = 2 * t
            wait_gather(j0, rows0, sem0)
            scatter(j0, rows0)

            @pl.when(j0 + 2 < NCHUNK)
            def _():
                start_gather(j0 + 2, rows0, sem0)

            wait_gather(j0 + 1, rows1, sem1)
            scatter(j0 + 1, rows1)

            @pl.when(j0 + 3 < NCHUNK)
            def _():
                start_gather(j0 + 3, rows1, sem1)

        wait_gather(NCHUNK - 1, rows0, sem0)
        scatter(NCHUNK - 1, rows0)

        plsc.subcore_barrier()

        def copy_out(chunk, nrows):
            base = pl.multiple_of(s * RPT + chunk * CH, 8)
            pltpu.sync_copy(acc_sh.at[pl.ds(base, nrows)],
                            zb.at[pl.ds(0, nrows)])
            pltpu.sync_copy(
                zb.at[pl.ds(0, nrows)],
                out_hbm.at[pl.ds(pl.multiple_of(c * N + s * RPT
                                                + chunk * CH, 8), nrows)])

        @pl.when(s < NS - 1)
        def _():
            for t in range(5):
                copy_out(t, CH)

        @pl.when(s == NS - 1)
        def _():
            for t in range(3):
                copy_out(t, CH)
            copy_out(3, N - (NS - 1) * RPT - 3 * CH)

        @pl.loop(0, CH)
        def _(r):
            for j in range(HH // 16):
                zb[r, pl.ds(j * 16, 16)] = zero16


@functools.cache
def _sc_agg_kernel():
    return pl.kernel(
        _sc_agg_body,
        out_type=(jax.ShapeDtypeStruct((NC * N, HH), jnp.float32),
                  jax.ShapeDtypeStruct((NC * N, HH), jnp.float32)),
        mesh=_mesh(),
        compiler_params=pltpu.CompilerParams(use_tc_tiling_on_sc=False),
        scratch_types=[
            pltpu.VMEM((NCHUNK, K), jnp.int32),
            pltpu.VMEM((NCHUNK, K), jnp.int32),
            pltpu.VMEM((K, HH), jnp.float32),
            pltpu.VMEM((K, HH), jnp.float32),
            pltpu.VMEM((RPT // 5, HH), jnp.float32),
            pltpu.VMEM_SHARED((NP_, HH), jnp.float32),
            pltpu.SemaphoreType.DMA,
            pltpu.SemaphoreType.DMA,
        ],
    )


def _sc_agg(ga, gb, src3, dst3):
    return _sc_agg_kernel()(ga, gb, src3, dst3)


def _prep_body(x_ref, w_ref, dinv_ref, ga_ref, gb_ref):
    u = jnp.dot(x_ref[...], w_ref[...], preferred_element_type=jnp.float32)
    g = u * dinv_ref[...]
    ga_ref[...] = g[:, :HH]
    gb_ref[...] = g[:, HH:]


def _tc_prep(x, W1, dinv):
    return pl.pallas_call(
        _prep_body,
        grid=(NB,),
        in_specs=[
            pl.BlockSpec((BN, D), lambda i: (i, 0)),
            pl.BlockSpec((D, H), lambda i: (0, 0)),
            pl.BlockSpec((BN, 1), lambda i: (i, 0)),
        ],
        out_specs=[pl.BlockSpec((BN, HH), lambda i: (i, 0)),
                   pl.BlockSpec((BN, HH), lambda i: (i, 0))],
        out_shape=[jax.ShapeDtypeStruct((N, HH), jnp.float32),
                   jax.ShapeDtypeStruct((N, HH), jnp.float32)],
    )(x, W1, dinv)


def _gather_z(pa0, pa1, pb0, pb1, ga, gb, dinv, b):
    ph = jnp.concatenate([pa0[...] + pa1[...] + ga[...],
                          pb0[...] + pb1[...] + gb[...]], axis=1)
    return jnp.maximum(dinv[...] * ph + b[...], 0.0)


def _mid_body(pa0, pa1, pb0, pb1, ga, gb, dinv_ref, b_ref, w_ref,
              oa_ref, ob_ref):
    z = _gather_z(pa0, pa1, pb0, pb1, ga, gb, dinv_ref, b_ref)
    u = jnp.dot(z, w_ref[...], preferred_element_type=jnp.float32)
    g = u * dinv_ref[...]
    oa_ref[...] = g[:, :HH]
    ob_ref[...] = g[:, HH:]


def _tc_mid(aa, ab, ga, gb, dinv, b, W):
    half = lambda off: pl.BlockSpec((BN, HH), lambda i, off=off: (i + off, 0))
    return pl.pallas_call(
        _mid_body,
        grid=(NB,),
        in_specs=[
            half(0), half(NB), half(0), half(NB),
            pl.BlockSpec((BN, HH), lambda i: (i, 0)),
            pl.BlockSpec((BN, HH), lambda i: (i, 0)),
            pl.BlockSpec((BN, 1), lambda i: (i, 0)),
            pl.BlockSpec((1, H), lambda i: (0, 0)),
            pl.BlockSpec((H, H), lambda i: (0, 0)),
        ],
        out_specs=[pl.BlockSpec((BN, HH), lambda i: (i, 0)),
                   pl.BlockSpec((BN, HH), lambda i: (i, 0))],
        out_shape=[jax.ShapeDtypeStruct((N, HH), jnp.float32),
                   jax.ShapeDtypeStruct((N, HH), jnp.float32)],
    )(aa, aa, ab, ab, ga, gb, dinv, b, W)


def _final_body(pa0, pa1, pb0, pb1, ga, gb, dinv_ref, b_ref, batch_ref,
                wo_ref, bo_ref, o_ref, acc, cnt):
    i = pl.program_id(0)

    @pl.when(i == 0)
    def _():
        acc[...] = jnp.zeros_like(acc)
        cnt[...] = jnp.zeros_like(cnt)

    z = _gather_z(pa0, pa1, pb0, pb1, ga, gb, dinv_ref, b_ref)
    q = jnp.dot(z, wo_ref[...], preferred_element_type=jnp.float32)
    gids = lax.broadcasted_iota(jnp.int32, (1, G), 1)
    m = (batch_ref[...] == gids).astype(jnp.float32)
    dn = (((0,), (0,)), ((), ()))
    acc[...] += lax.dot_general(m, q, dn, preferred_element_type=jnp.float32)
    cnt[...] += lax.dot_general(m, jnp.ones((BN, 1), jnp.float32), dn,
                                preferred_element_type=jnp.float32)

    @pl.when(i == NB - 1)
    def _():
        o_ref[...] = acc[...] / jnp.maximum(cnt[...], 1.0) + bo_ref[...]


def _tc_final(aa, ab, ga, gb, dinv, b, batch2d, Wo, bo2d):
    half = lambda off: pl.BlockSpec((BN, HH), lambda i, off=off: (i + off, 0))
    return pl.pallas_call(
        _final_body,
        grid=(NB,),
        in_specs=[
            half(0), half(NB), half(0), half(NB),
            pl.BlockSpec((BN, HH), lambda i: (i, 0)),
            pl.BlockSpec((BN, HH), lambda i: (i, 0)),
            pl.BlockSpec((BN, 1), lambda i: (i, 0)),
            pl.BlockSpec((1, H), lambda i: (0, 0)),
            pl.BlockSpec((BN, 1), lambda i: (i, 0)),
            pl.BlockSpec((H, 1), lambda i: (0, 0)),
            pl.BlockSpec((1, 1), lambda i: (0, 0)),
        ],
        out_specs=pl.BlockSpec((G, 1), lambda i: (0, 0)),
        out_shape=jax.ShapeDtypeStruct((G, 1), jnp.float32),
        scratch_shapes=[
            pltpu.VMEM((G, 1), jnp.float32),
            pltpu.VMEM((G, 1), jnp.float32),
        ],
    )(aa, aa, ab, ab, ga, gb, dinv, b, batch2d, Wo, bo2d)


def kernel(x, edge_index, batch, W1, b1, W2, b2, Wo, bo):
    src3 = edge_index[0].reshape(NW, NCHUNK, K)
    dst3 = edge_index[1].reshape(NW, NCHUNK, K)

    degp = _sc_deg(dst3)
    dinv = lax.rsqrt(degp[:N] + degp[N:] + 1.0).reshape(N, 1)

    g1a, g1b = _tc_prep(x, W1, dinv)
    a1a, a1b = _sc_agg(g1a, g1b, src3, dst3)
    g2a, g2b = _tc_mid(a1a, a1b, g1a, g1b, dinv, b1.reshape(1, H), W2)
    a2a, a2b = _sc_agg(g2a, g2b, src3, dst3)
    out = _tc_final(a2a, a2b, g2a, g2b, dinv, b2.reshape(1, H),
                    batch.reshape(N, 1), Wo, bo.reshape(1, 1))
    return out.reshape(-1)


# speedup vs baseline: 27.5534x; 1.2019x over previous
import functools

import jax
import jax.numpy as jnp
from jax import lax
from jax.experimental import pallas as pl
from jax.experimental.pallas import tpu as pltpu
from jax.experimental.pallas import tpu_sc as plsc

N = 10000
E = 320000
D = 128
H = 128
G = 128

NC = 2
NS = 16
NW = NC * NS
EPW = E // NW
K = 80
NCHUNK = EPW // K
NP_ = 10240
RPT = NP_ // NS

BN = 1000
NB = N // BN

def _mesh():
    return plsc.VectorSubcoreMesh(core_axis_name="c", subcore_axis_name="s",
                                  num_cores=NC, num_subcores=NS)


def _sc_deg_body(dst_hbm, out_hbm, dstb, ones_v, zb, deg_sh, sem):
    c = lax.axis_index("c")
    s = lax.axis_index("s")
    w = s * NC + c

    one16 = jnp.ones((16,), jnp.float32)
    zero16 = jnp.zeros((16,), jnp.float32)
    for j in range(K // 16):
        ones_v[pl.ds(j * 16, 16)] = one16

    @pl.loop(0, 1000 // 16)
    def _(i):
        zb[pl.ds(i * 16, 16)] = zero16

    @pl.when(s < 10)
    def _():
        pltpu.sync_copy(zb, deg_sh.at[pl.ds(pl.multiple_of(s * 1000, 8), 1000)])

    pltpu.sync_copy(dst_hbm.at[w], dstb)
    plsc.subcore_barrier()

    AHEAD = 4

    @pl.loop(0, NCHUNK)
    def _(j):
        pltpu.async_copy(ones_v, deg_sh.at[dstb.at[j]], sem, add=True)

        @pl.when(j >= AHEAD)
        def _():
            pltpu.make_async_copy(ones_v, deg_sh.at[dstb.at[0]], sem).wait()

    for _ in range(AHEAD):
        pltpu.make_async_copy(ones_v, deg_sh.at[dstb.at[0]], sem).wait()

    plsc.subcore_barrier()

    @pl.when(s < 10)
    def _():
        pltpu.sync_copy(deg_sh.at[pl.ds(pl.multiple_of(s * 1000, 8), 1000)],
                        zb)
        pltpu.sync_copy(zb,
                        out_hbm.at[pl.ds(pl.multiple_of(c * N + s * 1000, 8),
                                         1000)])


@functools.cache
def _sc_deg_kernel():
    return pl.kernel(
        _sc_deg_body,
        out_type=jax.ShapeDtypeStruct((NC * N,), jnp.float32),
        mesh=_mesh(),
        compiler_params=pltpu.CompilerParams(use_tc_tiling_on_sc=False),
        scratch_types=[
            pltpu.VMEM((NCHUNK, K), jnp.int32),
            pltpu.VMEM((K,), jnp.float32),
            pltpu.VMEM((1000,), jnp.float32),
            pltpu.VMEM_SHARED((N,), jnp.float32),
            pltpu.SemaphoreType.DMA,
        ],
    )


def _sc_deg(dst3):
    return _sc_deg_kernel()(dst3)


HH = H // 2


def _sc_agg_body(ga_hbm, gb_hbm, src_hbm, dst_hbm, outa_hbm, outb_hbm,
                 srcb, dstb, bufs0, bufs1, bufs2, bufs3, zb0, zb1, acc_sh,
                 gsems, ssems, zsem):
    c = lax.axis_index("c")
    s = lax.axis_index("s")
    w = s * NC + c
    bufs = (bufs0, bufs1, bufs2, bufs3)

    zero16 = jnp.zeros((16,), jnp.float32)
    CH = RPT // 5

    def fill_zero(zb):
        @pl.loop(0, CH)
        def _(r):
            for j in range(HH // 16):
                zb[r, pl.ds(j * 16, 16)] = zero16

    fill_zero(zb0)
    fill_zero(zb1)

    pltpu.sync_copy(src_hbm.at[w], srcb)
    pltpu.sync_copy(dst_hbm.at[w], dstb)

    def zero_acc():
        for t in range(5):
            pltpu.async_copy(
                zb0,
                acc_sh.at[pl.ds(pl.multiple_of(s * RPT + t * CH, 8), CH)],
                zsem)
        for t in range(5):
            pltpu.make_async_copy(
                zb0,
                acc_sh.at[pl.ds(pl.multiple_of(s * RPT + t * CH, 8), CH)],
                zsem).wait()

    for g_hbm, out_hbm in ((ga_hbm, outa_hbm), (gb_hbm, outb_hbm)):
        zero_acc()
        plsc.subcore_barrier()

        def g_start(j, l):
            pltpu.async_copy(g_hbm.at[srcb.at[j]], bufs[l], gsems.at[l])

        def g_wait(j, l):
            pltpu.make_async_copy(g_hbm.at[srcb.at[j]], bufs[l],
                                  gsems.at[l]).wait()

        def s_start(j, l):
            pltpu.async_copy(bufs[l], acc_sh.at[dstb.at[j]], ssems.at[l],
                             add=True)

        def s_wait(j, l):
            pltpu.make_async_copy(bufs[l], acc_sh.at[dstb.at[j]],
                                  ssems.at[l]).wait()

        for l in range(4):
            g_start(l, l)

        @pl.loop(0, (NCHUNK - 1) // 4)
        def _(t):
            j0 = 4 * t
            for l in range(4):
                g_wait(j0 + l, l)
                s_start(j0 + l, l)
            for l in range(4):
                @pl.when(j0 + l + 4 < NCHUNK)
                def _(l=l):
                    s_wait(j0 + l, l)
                    g_start(j0 + l + 4, l)

        g_wait(NCHUNK - 1, 0)
        s_start(NCHUNK - 1, 0)
        for l in range(1, 4):
            s_wait(NCHUNK - 5 + l, l)
        s_wait(NCHUNK - 1, 0)

        plsc.subcore_barrier()

        zbs = (zb0, zb1)

        def stage_in(chunk, nrows, l):
            base = pl.multiple_of(s * RPT + chunk * CH, 8)
            pltpu.async_copy(acc_sh.at[pl.ds(base, nrows)],
                             zbs[l].at[pl.ds(0, nrows)], gsems.at[l])

        def stage_out(chunk, nrows, l):
            base = pl.multiple_of(s * RPT + chunk * CH, 8)
            pltpu.make_async_copy(acc_sh.at[pl.ds(base, nrows)],
                                  zbs[l].at[pl.ds(0, nrows)],
                                  gsems.at[l]).wait()
            pltpu.sync_copy(
                zbs[l].at[pl.ds(0, nrows)],
                out_hbm.at[pl.ds(pl.multiple_of(c * N + s * RPT
                                                + chunk * CH, 8), nrows)])

        @pl.when(s < NS - 1)
        def _():
            stage_in(0, CH, 0)
            stage_in(1, CH, 1)
            stage_out(0, CH, 0)
            stage_in(2, CH, 0)
            stage_out(1, CH, 1)
            stage_in(3, CH, 1)
            stage_out(2, CH, 0)
            stage_in(4, CH, 0)
            stage_out(3, CH, 1)
            stage_out(4, CH, 0)

        @pl.when(s == NS - 1)
        def _():
            TAIL = N - (NS - 1) * RPT - 3 * CH
            stage_in(0, CH, 0)
            stage_in(1, CH, 1)
            stage_out(0, CH, 0)
            stage_in(2, CH, 0)
            stage_out(1, CH, 1)
            stage_in(3, TAIL, 1)
            stage_out(2, CH, 0)
            stage_out(3, TAIL, 1)

        fill_zero(zb0)
        fill_zero(zb1)


@functools.cache
def _sc_agg_kernel():
    return pl.kernel(
        _sc_agg_body,
        out_type=(jax.ShapeDtypeStruct((NC * N, HH), jnp.float32),
                  jax.ShapeDtypeStruct((NC * N, HH), jnp.float32)),
        mesh=_mesh(),
        compiler_params=pltpu.CompilerParams(use_tc_tiling_on_sc=False),
        scratch_types=[
            pltpu.VMEM((NCHUNK, K), jnp.int32),
            pltpu.VMEM((NCHUNK, K), jnp.int32),
            pltpu.VMEM((K, HH), jnp.float32),
            pltpu.VMEM((K, HH), jnp.float32),
            pltpu.VMEM((K, HH), jnp.float32),
            pltpu.VMEM((K, HH), jnp.float32),
            pltpu.VMEM((RPT // 5, HH), jnp.float32),
            pltpu.VMEM((RPT // 5, HH), jnp.float32),
            pltpu.VMEM_SHARED((NP_, HH), jnp.float32),
            pltpu.SemaphoreType.DMA((4,)),
            pltpu.SemaphoreType.DMA((4,)),
            pltpu.SemaphoreType.DMA,
        ],
    )


def _sc_agg(ga, gb, src3, dst3):
    return _sc_agg_kernel()(ga, gb, src3, dst3)


def _prep_body(x_ref, w_ref, dinv_ref, ga_ref, gb_ref):
    u = jnp.dot(x_ref[...], w_ref[...], preferred_element_type=jnp.float32)
    g = u * dinv_ref[...]
    ga_ref[...] = g[:, :HH]
    gb_ref[...] = g[:, HH:]


def _tc_prep(x, W1, dinv):
    return pl.pallas_call(
        _prep_body,
        grid=(NB,),
        in_specs=[
            pl.BlockSpec((BN, D), lambda i: (i, 0)),
            pl.BlockSpec((D, H), lambda i: (0, 0)),
            pl.BlockSpec((BN, 1), lambda i: (i, 0)),
        ],
        out_specs=[pl.BlockSpec((BN, HH), lambda i: (i, 0)),
                   pl.BlockSpec((BN, HH), lambda i: (i, 0))],
        out_shape=[jax.ShapeDtypeStruct((N, HH), jnp.float32),
                   jax.ShapeDtypeStruct((N, HH), jnp.float32)],
    )(x, W1, dinv)


def _gather_z(pa0, pa1, pb0, pb1, ga, gb, dinv, b):
    ph = jnp.concatenate([pa0[...] + pa1[...] + ga[...],
                          pb0[...] + pb1[...] + gb[...]], axis=1)
    return jnp.maximum(dinv[...] * ph + b[...], 0.0)


def _mid_body(pa0, pa1, pb0, pb1, ga, gb, dinv_ref, b_ref, w_ref,
              oa_ref, ob_ref):
    z = _gather_z(pa0, pa1, pb0, pb1, ga, gb, dinv_ref, b_ref)
    u = jnp.dot(z, w_ref[...], preferred_element_type=jnp.float32)
    g = u * dinv_ref[...]
    oa_ref[...] = g[:, :HH]
    ob_ref[...] = g[:, HH:]


def _tc_mid(aa, ab, ga, gb, dinv, b, W):
    half = lambda off: pl.BlockSpec((BN, HH), lambda i, off=off: (i + off, 0))
    return pl.pallas_call(
        _mid_body,
        grid=(NB,),
        in_specs=[
            half(0), half(NB), half(0), half(NB),
            pl.BlockSpec((BN, HH), lambda i: (i, 0)),
            pl.BlockSpec((BN, HH), lambda i: (i, 0)),
            pl.BlockSpec((BN, 1), lambda i: (i, 0)),
            pl.BlockSpec((1, H), lambda i: (0, 0)),
            pl.BlockSpec((H, H), lambda i: (0, 0)),
        ],
        out_specs=[pl.BlockSpec((BN, HH), lambda i: (i, 0)),
                   pl.BlockSpec((BN, HH), lambda i: (i, 0))],
        out_shape=[jax.ShapeDtypeStruct((N, HH), jnp.float32),
                   jax.ShapeDtypeStruct((N, HH), jnp.float32)],
    )(aa, aa, ab, ab, ga, gb, dinv, b, W)


def _final_body(pa0, pa1, pb0, pb1, ga, gb, dinv_ref, b_ref, batch_ref,
                wo_ref, bo_ref, o_ref, acc, cnt):
    i = pl.program_id(0)

    @pl.when(i == 0)
    def _():
        acc[...] = jnp.zeros_like(acc)
        cnt[...] = jnp.zeros_like(cnt)

    z = _gather_z(pa0, pa1, pb0, pb1, ga, gb, dinv_ref, b_ref)
    q = jnp.dot(z, wo_ref[...], preferred_element_type=jnp.float32)
    gids = lax.broadcasted_iota(jnp.int32, (1, G), 1)
    m = (batch_ref[...] == gids).astype(jnp.float32)
    dn = (((0,), (0,)), ((), ()))
    acc[...] += lax.dot_general(m, q, dn, preferred_element_type=jnp.float32)
    cnt[...] += lax.dot_general(m, jnp.ones((BN, 1), jnp.float32), dn,
                                preferred_element_type=jnp.float32)

    @pl.when(i == NB - 1)
    def _():
        o_ref[...] = acc[...] / jnp.maximum(cnt[...], 1.0) + bo_ref[...]


def _tc_final(aa, ab, ga, gb, dinv, b, batch2d, Wo, bo2d):
    half = lambda off: pl.BlockSpec((BN, HH), lambda i, off=off: (i + off, 0))
    return pl.pallas_call(
        _final_body,
        grid=(NB,),
        in_specs=[
            half(0), half(NB), half(0), half(NB),
            pl.BlockSpec((BN, HH), lambda i: (i, 0)),
            pl.BlockSpec((BN, HH), lambda i: (i, 0)),
            pl.BlockSpec((BN, 1), lambda i: (i, 0)),
            pl.BlockSpec((1, H), lambda i: (0, 0)),
            pl.BlockSpec((BN, 1), lambda i: (i, 0)),
            pl.BlockSpec((H, 1), lambda i: (0, 0)),
            pl.BlockSpec((1, 1), lambda i: (0, 0)),
        ],
        out_specs=pl.BlockSpec((G, 1), lambda i: (0, 0)),
        out_shape=jax.ShapeDtypeStruct((G, 1), jnp.float32),
        scratch_shapes=[
            pltpu.VMEM((G, 1), jnp.float32),
            pltpu.VMEM((G, 1), jnp.float32),
        ],
    )(aa, aa, ab, ab, ga, gb, dinv, b, batch2d, Wo, bo2d)


def kernel(x, edge_index, batch, W1, b1, W2, b2, Wo, bo):
    src3 = edge_index[0].reshape(NW, NCHUNK, K)
    dst3 = edge_index[1].reshape(NW, NCHUNK, K)

    degp = _sc_deg(dst3)
    dinv = lax.rsqrt(degp[:N] + degp[N:] + 1.0).reshape(N, 1)

    g1a, g1b = _tc_prep(x, W1, dinv)
    a1a, a1b = _sc_agg(g1a, g1b, src3, dst3)
    g2a, g2b = _tc_mid(a1a, a1b, g1a, g1b, dinv, b1.reshape(1, H), W2)
    a2a, a2b = _sc_agg(g2a, g2b, src3, dst3)
    out = _tc_final(a2a, a2b, g2a, g2b, dinv, b2.reshape(1, H),
                    batch.reshape(N, 1), Wo, bo.reshape(1, 1))
    return out.reshape(-1)


# speedup vs baseline: 31.7242x; 1.1514x over previous
import functools

import jax
import jax.numpy as jnp
from jax import lax
from jax.experimental import pallas as pl
from jax.experimental.pallas import tpu as pltpu
from jax.experimental.pallas import tpu_sc as plsc

N = 10000
E = 320000
D = 128
H = 128
G = 128

NC = 2
NS = 16
NW = NC * NS
EPW = E // NW
K = 80
NCHUNK = EPW // K
NP_ = 10240
RPT = NP_ // NS

BN = 1000
NB = N // BN

def _mesh():
    return plsc.VectorSubcoreMesh(core_axis_name="c", subcore_axis_name="s",
                                  num_cores=NC, num_subcores=NS)


def _sc_deg_body(dst_hbm, out_hbm, dstb, ones_v, zb, deg_sh, sem):
    c = lax.axis_index("c")
    s = lax.axis_index("s")
    w = s * NC + c

    one16 = jnp.ones((16,), jnp.float32)
    zero16 = jnp.zeros((16,), jnp.float32)
    for j in range(K // 16):
        ones_v[pl.ds(j * 16, 16)] = one16

    @pl.loop(0, 1000 // 16)
    def _(i):
        zb[pl.ds(i * 16, 16)] = zero16

    @pl.when(s < 10)
    def _():
        pltpu.sync_copy(zb, deg_sh.at[pl.ds(pl.multiple_of(s * 1000, 8), 1000)])

    pltpu.sync_copy(dst_hbm.at[w], dstb)
    plsc.subcore_barrier()

    AHEAD = 4

    @pl.loop(0, NCHUNK)
    def _(j):
        pltpu.async_copy(ones_v, deg_sh.at[dstb.at[j]], sem, add=True)

        @pl.when(j >= AHEAD)
        def _():
            pltpu.make_async_copy(ones_v, deg_sh.at[dstb.at[0]], sem).wait()

    for _ in range(AHEAD):
        pltpu.make_async_copy(ones_v, deg_sh.at[dstb.at[0]], sem).wait()

    plsc.subcore_barrier()

    @pl.when(s < 10)
    def _():
        pltpu.sync_copy(deg_sh.at[pl.ds(pl.multiple_of(s * 1000, 8), 1000)],
                        zb)
        pltpu.sync_copy(zb,
                        out_hbm.at[pl.ds(pl.multiple_of(c * N + s * 1000, 8),
                                         1000)])


@functools.cache
def _sc_deg_kernel():
    return pl.kernel(
        _sc_deg_body,
        out_type=jax.ShapeDtypeStruct((NC * N,), jnp.float32),
        mesh=_mesh(),
        compiler_params=pltpu.CompilerParams(use_tc_tiling_on_sc=False),
        scratch_types=[
            pltpu.VMEM((NCHUNK, K), jnp.int32),
            pltpu.VMEM((K,), jnp.float32),
            pltpu.VMEM((1000,), jnp.float32),
            pltpu.VMEM_SHARED((N,), jnp.float32),
            pltpu.SemaphoreType.DMA,
        ],
    )


def _sc_deg(dst3):
    return _sc_deg_kernel()(dst3)


HH = H // 2


def _sc_agg_body(g_hbm, src_hbm, dst_hbm, out_hbm,
                 srcb, srcb2, dstb, bufs0, bufs1, bufs2, bufs3, zb0, zb1,
                 acc_sh, gsems, ssems, zsem):
    c = lax.axis_index("c")
    s = lax.axis_index("s")
    w = s * NC + c
    bufs = (bufs0, bufs1, bufs2, bufs3)

    zero16 = jnp.zeros((16,), jnp.float32)
    CH = RPT // 5

    def fill_zero(zb):
        @pl.loop(0, CH)
        def _(r):
            for j in range(HH // 16):
                zb[r, pl.ds(j * 16, 16)] = zero16

    fill_zero(zb0)
    fill_zero(zb1)

    pltpu.sync_copy(src_hbm.at[w], srcb)
    pltpu.sync_copy(dst_hbm.at[w], dstb)

    def zero_acc():
        for t in range(5):
            pltpu.async_copy(
                zb0,
                acc_sh.at[pl.ds(pl.multiple_of(s * RPT + t * CH, 8), CH)],
                zsem)
        for t in range(5):
            pltpu.make_async_copy(
                zb0,
                acc_sh.at[pl.ds(pl.multiple_of(s * RPT + t * CH, 8), CH)],
                zsem).wait()

    for p in range(2):
        @pl.loop(0, NCHUNK)
        def _(j):
            for q in range(K // 16):
                srcb2[j, pl.ds(q * 16, 16)] = (
                    srcb[j, pl.ds(q * 16, 16)] * 2 + p)

        zero_acc()
        plsc.subcore_barrier()

        def g_start(j, l):
            pltpu.async_copy(g_hbm.at[srcb2.at[j]], bufs[l], gsems.at[l])

        def g_wait(j, l):
            pltpu.make_async_copy(g_hbm.at[srcb2.at[j]], bufs[l],
                                  gsems.at[l]).wait()

        def s_start(j, l):
            pltpu.async_copy(bufs[l], acc_sh.at[dstb.at[j]], ssems.at[l],
                             add=True)

        def s_wait(j, l):
            pltpu.make_async_copy(bufs[l], acc_sh.at[dstb.at[j]],
                                  ssems.at[l]).wait()

        for l in range(4):
            g_start(l, l)

        @pl.loop(0, (NCHUNK - 1) // 4)
        def _(t):
            j0 = 4 * t
            for l in range(4):
                g_wait(j0 + l, l)
                s_start(j0 + l, l)
            for l in range(4):
                @pl.when(j0 + l + 4 < NCHUNK)
                def _(l=l):
                    s_wait(j0 + l, l)
                    g_start(j0 + l + 4, l)

        g_wait(NCHUNK - 1, 0)
        s_start(NCHUNK - 1, 0)
        for l in range(1, 4):
            s_wait(NCHUNK - 5 + l, l)
        s_wait(NCHUNK - 1, 0)

        plsc.subcore_barrier()

        zbs = (zb0, zb1)

        def stage_in(chunk, nrows, l):
            base = pl.multiple_of(s * RPT + chunk * CH, 8)
            pltpu.async_copy(acc_sh.at[pl.ds(base, nrows)],
                             zbs[l].at[pl.ds(0, nrows)], gsems.at[l])

        def stage_out(chunk, nrows, l):
            base = pl.multiple_of(s * RPT + chunk * CH, 8)
            pltpu.make_async_copy(acc_sh.at[pl.ds(base, nrows)],
                                  zbs[l].at[pl.ds(0, nrows)],
                                  gsems.at[l]).wait()
            pltpu.sync_copy(
                zbs[l].at[pl.ds(0, nrows)],
                out_hbm.at[pl.ds(pl.multiple_of(c * N + s * RPT
                                                + chunk * CH, 8), nrows),
                           pl.ds(p * HH, HH)])

        @pl.when(s < NS - 1)
        def _():
            stage_in(0, CH, 0)
            stage_in(1, CH, 1)
            stage_out(0, CH, 0)
            stage_in(2, CH, 0)
            stage_out(1, CH, 1)
            stage_in(3, CH, 1)
            stage_out(2, CH, 0)
            stage_in(4, CH, 0)
            stage_out(3, CH, 1)
            stage_out(4, CH, 0)

        @pl.when(s == NS - 1)
        def _():
            TAIL = N - (NS - 1) * RPT - 3 * CH
            stage_in(0, CH, 0)
            stage_in(1, CH, 1)
            stage_out(0, CH, 0)
            stage_in(2, CH, 0)
            stage_out(1, CH, 1)
            stage_in(3, TAIL, 1)
            stage_out(2, CH, 0)
            stage_out(3, TAIL, 1)

        fill_zero(zb0)
        fill_zero(zb1)


@functools.cache
def _sc_agg_kernel():
    return pl.kernel(
        _sc_agg_body,
        out_type=jax.ShapeDtypeStruct((NC * N, H), jnp.float32),
        mesh=_mesh(),
        compiler_params=pltpu.CompilerParams(use_tc_tiling_on_sc=False),
        scratch_types=[
            pltpu.VMEM((NCHUNK, K), jnp.int32),
            pltpu.VMEM((NCHUNK, K), jnp.int32),
            pltpu.VMEM((NCHUNK, K), jnp.int32),
            pltpu.VMEM((K, HH), jnp.float32),
            pltpu.VMEM((K, HH), jnp.float32),
            pltpu.VMEM((K, HH), jnp.float32),
            pltpu.VMEM((K, HH), jnp.float32),
            pltpu.VMEM((RPT // 5, HH), jnp.float32),
            pltpu.VMEM((RPT // 5, HH), jnp.float32),
            pltpu.VMEM_SHARED((NP_, HH), jnp.float32),
            pltpu.SemaphoreType.DMA((4,)),
            pltpu.SemaphoreType.DMA((4,)),
            pltpu.SemaphoreType.DMA,
        ],
    )


def _sc_agg(g, src3, dst3):
    return _sc_agg_kernel()(g.reshape(NC * N, HH), src3, dst3)


def _prep_body(x_ref, w_ref, dinv_ref, g_ref):
    u = jnp.dot(x_ref[...], w_ref[...], preferred_element_type=jnp.float32)
    g_ref[...] = u * dinv_ref[...]


def _tc_prep(x, W1, dinvb):
    return pl.pallas_call(
        _prep_body,
        grid=(NB,),
        in_specs=[
            pl.BlockSpec((BN, D), lambda i: (i, 0)),
            pl.BlockSpec((D, H), lambda i: (0, 0)),
            pl.BlockSpec((BN, H), lambda i: (i, 0)),
        ],
        out_specs=pl.BlockSpec((BN, H), lambda i: (i, 0)),
        out_shape=jax.ShapeDtypeStruct((N, H), jnp.float32),
    )(x, W1, dinvb)


def _mid_body(p0_ref, p1_ref, g_ref, dinv_ref, b_ref, w_ref, o_ref):
    z = dinv_ref[...] * (p0_ref[...] + p1_ref[...] + g_ref[...]) + b_ref[...]
    z = jnp.maximum(z, 0.0)
    u = jnp.dot(z, w_ref[...], preferred_element_type=jnp.float32)
    o_ref[...] = u * dinv_ref[...]


def _tc_mid(agg, g, dinvb, b, W):
    return pl.pallas_call(
        _mid_body,
        grid=(NB,),
        in_specs=[
            pl.BlockSpec((BN, H), lambda i: (i, 0)),
            pl.BlockSpec((BN, H), lambda i: (i + NB, 0)),
            pl.BlockSpec((BN, H), lambda i: (i, 0)),
            pl.BlockSpec((BN, H), lambda i: (i, 0)),
            pl.BlockSpec((1, H), lambda i: (0, 0)),
            pl.BlockSpec((H, H), lambda i: (0, 0)),
        ],
        out_specs=pl.BlockSpec((BN, H), lambda i: (i, 0)),
        out_shape=jax.ShapeDtypeStruct((N, H), jnp.float32),
    )(agg, agg, g, dinvb, b, W)


def _final_body(p0_ref, p1_ref, g_ref, dinv_ref, b_ref, batch_ref, wo_ref,
                bo_ref, o_ref, acc, cnt):
    i = pl.program_id(0)

    @pl.when(i == 0)
    def _():
        acc[...] = jnp.zeros_like(acc)
        cnt[...] = jnp.zeros_like(cnt)

    z = dinv_ref[...] * (p0_ref[...] + p1_ref[...] + g_ref[...]) + b_ref[...]
    z = jnp.maximum(z, 0.0)
    q = jnp.dot(z, wo_ref[...], preferred_element_type=jnp.float32)
    gids = lax.broadcasted_iota(jnp.int32, (BN, G), 1)
    m = (batch_ref[...] == gids).astype(jnp.float32)
    dn = (((0,), (0,)), ((), ()))
    acc[...] += lax.dot_general(m, q, dn, preferred_element_type=jnp.float32)
    cnt[...] += lax.dot_general(m, jnp.ones((BN, 1), jnp.float32), dn,
                                preferred_element_type=jnp.float32)

    @pl.when(i == NB - 1)
    def _():
        o_ref[...] = acc[...] / jnp.maximum(cnt[...], 1.0) + bo_ref[...]


def _tc_final(agg, g, dinvb, b, batchb, Wo, bo2d):
    return pl.pallas_call(
        _final_body,
        grid=(NB,),
        in_specs=[
            pl.BlockSpec((BN, H), lambda i: (i, 0)),
            pl.BlockSpec((BN, H), lambda i: (i + NB, 0)),
            pl.BlockSpec((BN, H), lambda i: (i, 0)),
            pl.BlockSpec((BN, H), lambda i: (i, 0)),
            pl.BlockSpec((1, H), lambda i: (0, 0)),
            pl.BlockSpec((BN, G), lambda i: (i, 0)),
            pl.BlockSpec((H, 1), lambda i: (0, 0)),
            pl.BlockSpec((1, 1), lambda i: (0, 0)),
        ],
        out_specs=pl.BlockSpec((G, 1), lambda i: (0, 0)),
        out_shape=jax.ShapeDtypeStruct((G, 1), jnp.float32),
        scratch_shapes=[
            pltpu.VMEM((G, 1), jnp.float32),
            pltpu.VMEM((G, 1), jnp.float32),
        ],
    )(agg, agg, g, dinvb, b, batchb, Wo, bo2d)


def kernel(x, edge_index, batch, W1, b1, W2, b2, Wo, bo):
    src3 = edge_index[0].reshape(NW, NCHUNK, K)
    dst3 = edge_index[1].reshape(NW, NCHUNK, K)

    degp = _sc_deg(dst3)
    dinv = lax.rsqrt(degp[:N] + degp[N:] + 1.0)
    dinvb = jnp.broadcast_to(dinv[:, None], (N, H))
    batchb = jnp.broadcast_to(batch[:, None], (N, G))

    g1 = _tc_prep(x, W1, dinvb)
    a1 = _sc_agg(g1, src3, dst3)
    g2 = _tc_mid(a1, g1, dinvb, b1.reshape(1, H), W2)
    a2 = _sc_agg(g2, src3, dst3)
    out = _tc_final(a2, g2, dinvb, b2.reshape(1, H), batchb,
                    Wo, bo.reshape(1, 1))
    return out.reshape(-1)


# speedup vs baseline: 33.9709x; 1.0708x over previous
import functools

import jax
import jax.numpy as jnp
from jax import lax
from jax.experimental import pallas as pl
from jax.experimental.pallas import tpu as pltpu
from jax.experimental.pallas import tpu_sc as plsc

N = 10000
E = 320000
D = 128
H = 128
G = 128
HH = H // 2

NC = 2
NS = 16
NW = NC * NS

EU = E // 128
EUP = 2504
UPW = 80
UPA = 78

NP_ = 10240
RPT = NP_ // NS
CH = RPT // 5
NDP = 10240

BN = 1000
NB = N // BN


def _mesh():
    return plsc.VectorSubcoreMesh(core_axis_name="c", subcore_axis_name="s",
                                  num_cores=NC, num_subcores=NS)


def _sc_deg_body(ei_hbm, deg_hbm, src_hbm, dst_hbm,
                 ubuf, ones_v, zb, deg_sh, sem, wsem):
    c = lax.axis_index("c")
    s = lax.axis_index("s")
    w = s * NC + c
    base = w * UPW

    one16 = jnp.ones((16,), jnp.float32)
    zero16 = jnp.zeros((16,), jnp.float32)
    for j in range(128 // 16):
        ones_v[pl.ds(j * 16, 16)] = one16

    @pl.loop(0, 1024 // 16)
    def _(i):
        zb[pl.ds(i * 16, 16)] = zero16

    @pl.when(s < 10)
    def _():
        pltpu.sync_copy(zb, deg_sh.at[pl.ds(pl.multiple_of(s * 1024, 8),
                                            1024)])

    AHEAD = 8

    def stage_units(nu):
        @pl.loop(0, nu)
        def _(u):
            pltpu.async_copy(
                ei_hbm.at[:, pl.ds(pl.multiple_of(128 * (base + u), 128),
                                   128)],
                ubuf.at[u], sem)

            @pl.when(u >= AHEAD)
            def _():
                pltpu.make_async_copy(ei_hbm.at[:, pl.ds(0, 128)],
                                      ubuf.at[0], sem).wait()

        for _ in range(min(AHEAD, nu)):
            pltpu.make_async_copy(ei_hbm.at[:, pl.ds(0, 128)], ubuf.at[0],
                                  sem).wait()

    def histogram(nu):
        @pl.loop(0, nu)
        def _(u):
            pltpu.async_copy(ones_v, deg_sh.at[ubuf.at[u, 1]], wsem,
                             add=True)

            @pl.when(u >= AHEAD)
            def _():
                pltpu.make_async_copy(ones_v, deg_sh.at[ubuf.at[0, 1]],
                                      wsem).wait()

        for _ in range(min(AHEAD, nu)):
            pltpu.make_async_copy(ones_v, deg_sh.at[ubuf.at[0, 1]],
                                  wsem).wait()

    def write_units(nrows):
        pltpu.sync_copy(ubuf.at[pl.ds(0, nrows), 0],
                        src_hbm.at[pl.ds(pl.multiple_of(base, 8), nrows)])
        pltpu.sync_copy(ubuf.at[pl.ds(0, nrows), 1],
                        dst_hbm.at[pl.ds(pl.multiple_of(base, 8), nrows)])

    @pl.when(w < NW - 1)
    def _():
        stage_units(UPW)

    @pl.when(w == NW - 1)
    def _():
        stage_units(EU - (NW - 1) * UPW)

    plsc.subcore_barrier()

    @pl.when(w < NW - 1)
    def _():
        histogram(UPW)
        write_units(UPW)

    @pl.when(w == NW - 1)
    def _():
        histogram(EU - (NW - 1) * UPW)
        write_units(EUP - (NW - 1) * UPW)

    plsc.subcore_barrier()

    @pl.when(s < 10)
    def _():
        pltpu.sync_copy(deg_sh.at[pl.ds(pl.multiple_of(s * 1024, 8), 1024)],
                        zb)
        pltpu.sync_copy(zb,
                        deg_hbm.at[pl.ds(pl.multiple_of(
                            c * NDP + s * 1024, 8), 1024)])


@functools.cache
def _sc_deg_kernel():
    return pl.kernel(
        _sc_deg_body,
        out_type=(jax.ShapeDtypeStruct((NC * NDP,), jnp.float32),
                  jax.ShapeDtypeStruct((EUP, 128), jnp.int32),
                  jax.ShapeDtypeStruct((EUP, 128), jnp.int32)),
        mesh=_mesh(),
        compiler_params=pltpu.CompilerParams(use_tc_tiling_on_sc=True),
        scratch_types=[
            pltpu.VMEM((UPW, 2, 128), jnp.int32),
            pltpu.VMEM((128,), jnp.float32),
            pltpu.VMEM((1024,), jnp.float32),
            pltpu.VMEM_SHARED((NDP,), jnp.float32),
            pltpu.SemaphoreType.DMA,
            pltpu.SemaphoreType.DMA,
        ],
    )


def _sc_deg(edge_index):
    return _sc_deg_kernel()(edge_index)


def _sc_agg_body(g_hbm, src_hbm, dst_hbm, out_hbm,
                 srcb, srcb2, dstb, bufs0, bufs1, bufs2, bufs3, zb0, zb1, zc,
                 acc_sh, gsems, ssems, zsem):
    c = lax.axis_index("c")
    s = lax.axis_index("s")
    w = s * NC + c
    bufs = (bufs0, bufs1, bufs2, bufs3)

    zero16 = jnp.zeros((16,), jnp.float32)

    @pl.loop(0, CH)
    def _(r):
        for j in range(HH // 16):
            zc[r, pl.ds(j * 16, 16)] = zero16

    pltpu.sync_copy(src_hbm.at[pl.ds(w * UPA, UPA)],
                    srcb.at[pl.ds(0, UPA)])
    pltpu.sync_copy(dst_hbm.at[pl.ds(w * UPA, UPA)],
                    dstb.at[pl.ds(0, UPA)])

    @pl.when(w < 4)
    def _():
        pltpu.sync_copy(src_hbm.at[pl.ds(NW * UPA + w, 1)],
                        srcb.at[pl.ds(UPA, 1)])
        pltpu.sync_copy(dst_hbm.at[pl.ds(NW * UPA + w, 1)],
                        dstb.at[pl.ds(UPA, 1)])

    for t in range(5):
        pltpu.async_copy(
            zc, acc_sh.at[pl.ds(pl.multiple_of(s * RPT + t * CH, 8), CH)],
            zsem)
    for t in range(5):
        pltpu.make_async_copy(
            zc, acc_sh.at[pl.ds(pl.multiple_of(s * RPT + t * CH, 8), CH)],
            zsem).wait()

    for p in range(2):
        @pl.loop(0, UPA + 1)
        def _(j):
            for q in range(128 // 16):
                srcb2[j, pl.ds(q * 16, 16)] = (
                    srcb[j, pl.ds(q * 16, 16)] * 2 + p)

        plsc.subcore_barrier()

        def g_start(j, l):
            pltpu.async_copy(g_hbm.at[srcb2.at[j]], bufs[l], gsems.at[l])

        def g_wait(j, l):
            pltpu.make_async_copy(g_hbm.at[srcb2.at[j]], bufs[l],
                                  gsems.at[l]).wait()

        def s_start(j, l):
            pltpu.async_copy(bufs[l], acc_sh.at[dstb.at[j]], ssems.at[l],
                             add=True)

        def s_wait(j, l):
            pltpu.make_async_copy(bufs[l], acc_sh.at[dstb.at[j]],
                                  ssems.at[l]).wait()

        for l in range(4):
            g_start(l, l)

        @pl.loop(0, (UPA - 2) // 4)
        def _(t):
            j0 = 4 * t
            for l in range(4):
                g_wait(j0 + l, l)
                s_start(j0 + l, l)
            for l in range(4):
                @pl.when(j0 + l + 4 < UPA)
                def _(l=l):
                    s_wait(j0 + l, l)
                    g_start(j0 + l + 4, l)

        g_wait(UPA - 2, 0)
        s_start(UPA - 2, 0)
        g_wait(UPA - 1, 1)
        s_start(UPA - 1, 1)
        s_wait(UPA - 4, 2)
        s_wait(UPA - 3, 3)

        @pl.when(w < 4)
        def _():
            g_start(UPA, 2)
            g_wait(UPA, 2)
            s_start(UPA, 2)
            s_wait(UPA, 2)

        s_wait(UPA - 2, 0)
        s_wait(UPA - 1, 1)

        plsc.subcore_barrier()

        zbs = (zb0, zb1)

        def stage_in(chunk, nrows, l):
            base = pl.multiple_of(s * RPT + chunk * CH, 8)
            pltpu.async_copy(acc_sh.at[pl.ds(base, nrows)],
                             zbs[l].at[pl.ds(0, nrows)], gsems.at[l])

        def stage_out(chunk, nrows, l):
            base = pl.multiple_of(s * RPT + chunk * CH, 8)
            pltpu.make_async_copy(acc_sh.at[pl.ds(base, nrows)],
                                  zbs[l].at[pl.ds(0, nrows)],
                                  gsems.at[l]).wait()
            if p == 0:
                pltpu.async_copy(zc.at[pl.ds(0, nrows)],
                                 acc_sh.at[pl.ds(base, nrows)], zsem)
            pltpu.sync_copy(
                zbs[l].at[pl.ds(0, nrows)],
                out_hbm.at[pl.ds(pl.multiple_of(c * N + s * RPT
                                                + chunk * CH, 8), nrows),
                           pl.ds(p * HH, HH)])

        def drain_zero(chunk, nrows):
            if p == 0:
                base = pl.multiple_of(s * RPT + chunk * CH, 8)
                pltpu.make_async_copy(zc.at[pl.ds(0, nrows)],
                                      acc_sh.at[pl.ds(base, nrows)],
                                      zsem).wait()

        @pl.when(s < NS - 1)
        def _():
            stage_in(0, CH, 0)
            stage_in(1, CH, 1)
            stage_out(0, CH, 0)
            stage_in(2, CH, 0)
            stage_out(1, CH, 1)
            stage_in(3, CH, 1)
            stage_out(2, CH, 0)
            stage_in(4, CH, 0)
            stage_out(3, CH, 1)
            stage_out(4, CH, 0)
            for t in range(5):
                drain_zero(t, CH)

        @pl.when(s == NS - 1)
        def _():
            TAIL = N - (NS - 1) * RPT - 3 * CH
            stage_in(0, CH, 0)
            stage_in(1, CH, 1)
            stage_out(0, CH, 0)
            stage_in(2, CH, 0)
            stage_out(1, CH, 1)
            stage_in(3, TAIL, 1)
            stage_out(2, CH, 0)
            stage_out(3, TAIL, 1)
            for t in range(3):
                drain_zero(t, CH)
            drain_zero(3, TAIL)


@functools.cache
def _sc_agg_kernel():
    return pl.kernel(
        _sc_agg_body,
        out_type=jax.ShapeDtypeStruct((NC * N, H), jnp.float32),
        mesh=_mesh(),
        compiler_params=pltpu.CompilerParams(use_tc_tiling_on_sc=False),
        scratch_types=[
            pltpu.VMEM((UPA + 1, 128), jnp.int32),
            pltpu.VMEM((UPA + 1, 128), jnp.int32),
            pltpu.VMEM((UPA + 1, 128), jnp.int32),
            pltpu.VMEM((128, HH), jnp.float32),
            pltpu.VMEM((128, HH), jnp.float32),
            pltpu.VMEM((128, HH), jnp.float32),
            pltpu.VMEM((128, HH), jnp.float32),
            pltpu.VMEM((CH, HH), jnp.float32),
            pltpu.VMEM((CH, HH), jnp.float32),
            pltpu.VMEM((CH, HH), jnp.float32),
            pltpu.VMEM_SHARED((NP_, HH), jnp.float32),
            pltpu.SemaphoreType.DMA((4,)),
            pltpu.SemaphoreType.DMA((4,)),
            pltpu.SemaphoreType.DMA,
        ],
    )


def _sc_agg(g, src2, dst2):
    return _sc_agg_kernel()(g.reshape(NC * N, HH), src2, dst2)


def _prep_body(x_ref, w_ref, dinv_ref, g_ref):
    u = jnp.dot(x_ref[...], w_ref[...], preferred_element_type=jnp.float32)
    g_ref[...] = u * dinv_ref[...]


def _tc_prep(x, W1, dinvb):
    return pl.pallas_call(
        _prep_body,
        grid=(NB,),
        in_specs=[
            pl.BlockSpec((BN, D), lambda i: (i, 0)),
            pl.BlockSpec((D, H), lambda i: (0, 0)),
            pl.BlockSpec((BN, H), lambda i: (i, 0)),
        ],
        out_specs=pl.BlockSpec((BN, H), lambda i: (i, 0)),
        out_shape=jax.ShapeDtypeStruct((N, H), jnp.float32),
    )(x, W1, dinvb)


def _mid_body(p0_ref, p1_ref, g_ref, dinv_ref, b_ref, w_ref, o_ref):
    z = dinv_ref[...] * (p0_ref[...] + p1_ref[...] + g_ref[...]) + b_ref[...]
    z = jnp.maximum(z, 0.0)
    u = jnp.dot(z, w_ref[...], preferred_element_type=jnp.float32)
    o_ref[...] = u * dinv_ref[...]


def _tc_mid(agg, g, dinvb, b, W):
    return pl.pallas_call(
        _mid_body,
        grid=(NB,),
        in_specs=[
            pl.BlockSpec((BN, H), lambda i: (i, 0)),
            pl.BlockSpec((BN, H), lambda i: (i + NB, 0)),
            pl.BlockSpec((BN, H), lambda i: (i, 0)),
            pl.BlockSpec((BN, H), lambda i: (i, 0)),
            pl.BlockSpec((1, H), lambda i: (0, 0)),
            pl.BlockSpec((H, H), lambda i: (0, 0)),
        ],
        out_specs=pl.BlockSpec((BN, H), lambda i: (i, 0)),
        out_shape=jax.ShapeDtypeStruct((N, H), jnp.float32),
    )(agg, agg, g, dinvb, b, W)


def _final_body(p0_ref, p1_ref, g_ref, dinv_ref, b_ref, batch_ref, wo_ref,
                bo_ref, o_ref, acc, cnt):
    i = pl.program_id(0)

    @pl.when(i == 0)
    def _():
        acc[...] = jnp.zeros_like(acc)
        cnt[...] = jnp.zeros_like(cnt)

    z = dinv_ref[...] * (p0_ref[...] + p1_ref[...] + g_ref[...]) + b_ref[...]
    z = jnp.maximum(z, 0.0)
    q = jnp.dot(z, wo_ref[...], preferred_element_type=jnp.float32)
    gids = lax.broadcasted_iota(jnp.int32, (BN, G), 1)
    m = (batch_ref[...] == gids).astype(jnp.float32)
    dn = (((0,), (0,)), ((), ()))
    acc[...] += lax.dot_general(m, q, dn, preferred_element_type=jnp.float32)
    cnt[...] += lax.dot_general(m, jnp.ones((BN, 1), jnp.float32), dn,
                                preferred_element_type=jnp.float32)

    @pl.when(i == NB - 1)
    def _():
        o_ref[...] = acc[...] / jnp.maximum(cnt[...], 1.0) + bo_ref[...]


def _tc_final(agg, g, dinvb, b, batchb, Wo, bo2d):
    return pl.pallas_call(
        _final_body,
        grid=(NB,),
        in_specs=[
            pl.BlockSpec((BN, H), lambda i: (i, 0)),
            pl.BlockSpec((BN, H), lambda i: (i + NB, 0)),
            pl.BlockSpec((BN, H), lambda i: (i, 0)),
            pl.BlockSpec((BN, H), lambda i: (i, 0)),
            pl.BlockSpec((1, H), lambda i: (0, 0)),
            pl.BlockSpec((BN, G), lambda i: (i, 0)),
            pl.BlockSpec((H, 1), lambda i: (0, 0)),
            pl.BlockSpec((1, 1), lambda i: (0, 0)),
        ],
        out_specs=pl.BlockSpec((G, 1), lambda i: (0, 0)),
        out_shape=jax.ShapeDtypeStruct((G, 1), jnp.float32),
        scratch_shapes=[
            pltpu.VMEM((G, 1), jnp.float32),
            pltpu.VMEM((G, 1), jnp.float32),
        ],
    )(agg, agg, g, dinvb, b, batchb, Wo, bo2d)


def kernel(x, edge_index, batch, W1, b1, W2, b2, Wo, bo):
    degp, src2, dst2 = _sc_deg(edge_index)
    dinv = lax.rsqrt(degp[:N] + degp[NDP:NDP + N] + 1.0)
    dinvb = jnp.broadcast_to(dinv[:, None], (N, H))
    batchb = jnp.broadcast_to(batch[:, None], (N, G))

    g1 = _tc_prep(x, W1, dinvb)
    a1 = _sc_agg(g1, src2, dst2)
    g2 = _tc_mid(a1, g1, dinvb, b1.reshape(1, H), W2)
    a2 = _sc_agg(g2, src2, dst2)
    out = _tc_final(a2, g2, dinvb, b2.reshape(1, H), batchb,
                    Wo, bo.reshape(1, 1))
    return out.reshape(-1)


# speedup vs baseline: 35.3320x; 1.0401x over previous
import functools

import jax
import jax.numpy as jnp
from jax import lax
from jax.experimental import pallas as pl
from jax.experimental.pallas import tpu as pltpu
from jax.experimental.pallas import tpu_sc as plsc

N = 10000
E = 320000
D = 128
H = 128
G = 128
HH = H // 2

NC = 2
NS = 16
NW = NC * NS

EU = E // 128
EUP = 2504
UPW = 80
UPA = 78
UPT = 156

NP_ = 10240
RPT = NP_ // NS
CH = RPT // 5
NDP = 10240

BN = 1000
NB = N // BN


def _mesh():
    return plsc.VectorSubcoreMesh(core_axis_name="c", subcore_axis_name="s",
                                  num_cores=NC, num_subcores=NS)


def _sc_deg_body(ei_hbm, deg_hbm, src_hbm, dst_hbm,
                 ubuf, ones_v, zb, deg_sh, sem, wsem):
    c = lax.axis_index("c")
    s = lax.axis_index("s")
    w = s * NC + c
    base = w * UPW

    one16 = jnp.ones((16,), jnp.float32)
    zero16 = jnp.zeros((16,), jnp.float32)
    for j in range(128 // 16):
        ones_v[pl.ds(j * 16, 16)] = one16

    @pl.loop(0, 1024 // 16)
    def _(i):
        zb[pl.ds(i * 16, 16)] = zero16

    @pl.when(s < 10)
    def _():
        pltpu.sync_copy(zb, deg_sh.at[pl.ds(pl.multiple_of(s * 1024, 8),
                                            1024)])

    AHEAD = 8

    def stage_units(nu):
        @pl.loop(0, nu)
        def _(u):
            pltpu.async_copy(
                ei_hbm.at[:, pl.ds(pl.multiple_of(128 * (base + u), 128),
                                   128)],
                ubuf.at[u], sem)

            @pl.when(u >= AHEAD)
            def _():
                pltpu.make_async_copy(ei_hbm.at[:, pl.ds(0, 128)],
                                      ubuf.at[0], sem).wait()

        for _ in range(min(AHEAD, nu)):
            pltpu.make_async_copy(ei_hbm.at[:, pl.ds(0, 128)], ubuf.at[0],
                                  sem).wait()

    def histogram(nu):
        @pl.loop(0, nu)
        def _(u):
            pltpu.async_copy(ones_v, deg_sh.at[ubuf.at[u, 1]], wsem,
                             add=True)

            @pl.when(u >= AHEAD)
            def _():
                pltpu.make_async_copy(ones_v, deg_sh.at[ubuf.at[0, 1]],
                                      wsem).wait()

        for _ in range(min(AHEAD, nu)):
            pltpu.make_async_copy(ones_v, deg_sh.at[ubuf.at[0, 1]],
                                  wsem).wait()

    def write_units(nrows):
        pltpu.sync_copy(ubuf.at[pl.ds(0, nrows), 0],
                        src_hbm.at[pl.ds(pl.multiple_of(base, 8), nrows)])
        pltpu.sync_copy(ubuf.at[pl.ds(0, nrows), 1],
                        dst_hbm.at[pl.ds(pl.multiple_of(base, 8), nrows)])

    @pl.when(w < NW - 1)
    def _():
        stage_units(UPW)

    @pl.when(w == NW - 1)
    def _():
        stage_units(EU - (NW - 1) * UPW)

    plsc.subcore_barrier()

    @pl.when(w < NW - 1)
    def _():
        histogram(UPW)
        write_units(UPW)

    @pl.when(w == NW - 1)
    def _():
        histogram(EU - (NW - 1) * UPW)
        write_units(EUP - (NW - 1) * UPW)

    plsc.subcore_barrier()

    @pl.when(s < 10)
    def _():
        pltpu.sync_copy(deg_sh.at[pl.ds(pl.multiple_of(s * 1024, 8), 1024)],
                        zb)
        pltpu.sync_copy(zb,
                        deg_hbm.at[pl.ds(pl.multiple_of(
                            c * NDP + s * 1024, 8), 1024)])


@functools.cache
def _sc_deg_kernel():
    return pl.kernel(
        _sc_deg_body,
        out_type=(jax.ShapeDtypeStruct((NC * NDP,), jnp.float32),
                  jax.ShapeDtypeStruct((EUP, 128), jnp.int32),
                  jax.ShapeDtypeStruct((EUP, 128), jnp.int32)),
        mesh=_mesh(),
        compiler_params=pltpu.CompilerParams(use_tc_tiling_on_sc=True),
        scratch_types=[
            pltpu.VMEM((UPW, 2, 128), jnp.int32),
            pltpu.VMEM((128,), jnp.float32),
            pltpu.VMEM((1024,), jnp.float32),
            pltpu.VMEM_SHARED((NDP,), jnp.float32),
            pltpu.SemaphoreType.DMA,
            pltpu.SemaphoreType.DMA,
        ],
    )


def _sc_deg(edge_index):
    return _sc_deg_kernel()(edge_index)


def _sc_agg_body(g_hbm, src_hbm, dst_hbm, out_hbm,
                 srcb, dstb, bufs0, bufs1, bufs2, bufs3, zc,
                 acc_sh, gsems, ssems, zsem):
    c = lax.axis_index("c")
    s = lax.axis_index("s")
    bufs = (bufs0, bufs1, bufs2, bufs3)

    zero16 = jnp.zeros((16,), jnp.float32)

    @pl.loop(0, CH)
    def _(r):
        for j in range(HH // 16):
            zc[r, pl.ds(j * 16, 16)] = zero16

    pltpu.sync_copy(src_hbm.at[pl.ds(s * UPT, UPT)], srcb.at[pl.ds(0, UPT)])
    pltpu.sync_copy(dst_hbm.at[pl.ds(s * UPT, UPT)], dstb.at[pl.ds(0, UPT)])

    @pl.when(s < 4)
    def _():
        pltpu.sync_copy(src_hbm.at[pl.ds(NS * UPT + s, 1)],
                        srcb.at[pl.ds(UPT, 1)])
        pltpu.sync_copy(dst_hbm.at[pl.ds(NS * UPT + s, 1)],
                        dstb.at[pl.ds(UPT, 1)])

    for t in range(5):
        pltpu.async_copy(
            zc, acc_sh.at[pl.ds(pl.multiple_of(s * RPT + t * CH, 8), CH)],
            zsem)
    for t in range(5):
        pltpu.make_async_copy(
            zc, acc_sh.at[pl.ds(pl.multiple_of(s * RPT + t * CH, 8), CH)],
            zsem).wait()

    @pl.loop(0, UPT + 1)
    def _(j):
        for q in range(128 // 16):
            srcb[j, pl.ds(q * 16, 16)] = srcb[j, pl.ds(q * 16, 16)] * 2 + c

    plsc.subcore_barrier()

    def g_start(j, l):
        pltpu.async_copy(g_hbm.at[srcb.at[j]], bufs[l], gsems.at[l])

    def g_wait(j, l):
        pltpu.make_async_copy(g_hbm.at[srcb.at[j]], bufs[l],
                              gsems.at[l]).wait()

    def s_start(j, l):
        pltpu.async_copy(bufs[l], acc_sh.at[dstb.at[j]], ssems.at[l],
                         add=True)

    def s_wait(j, l):
        pltpu.make_async_copy(bufs[l], acc_sh.at[dstb.at[j]],
                              ssems.at[l]).wait()

    for l in range(4):
        g_start(l, l)

    @pl.loop(0, (UPT - 4) // 4)
    def _(t):
        j0 = 4 * t
        for l in range(4):
            g_wait(j0 + l, l)
            s_start(j0 + l, l)
        for l in range(4):
            s_wait(j0 + l, l)
            g_start(j0 + l + 4, l)

    for l in range(4):
        g_wait(UPT - 4 + l, l)
        s_start(UPT - 4 + l, l)
    s_wait(UPT - 4, 0)

    @pl.when(s < 4)
    def _():
        g_start(UPT, 0)
        g_wait(UPT, 0)
        s_start(UPT, 0)
        s_wait(UPT, 0)

    for l in range(1, 4):
        s_wait(UPT - 4 + l, l)

    plsc.subcore_barrier()

    zbs = (bufs0, bufs1)

    def stage_in(chunk, nrows, l):
        base = pl.multiple_of(s * RPT + chunk * CH, 8)
        pltpu.async_copy(acc_sh.at[pl.ds(base, nrows)],
                         zbs[l].at[pl.ds(0, nrows)], gsems.at[l])

    def stage_out(chunk, nrows, l):
        base = pl.multiple_of(s * RPT + chunk * CH, 8)
        pltpu.make_async_copy(acc_sh.at[pl.ds(base, nrows)],
                              zbs[l].at[pl.ds(0, nrows)],
                              gsems.at[l]).wait()
        pltpu.sync_copy(
            zbs[l].at[pl.ds(0, nrows)],
            out_hbm.at[pl.ds(pl.multiple_of(s * RPT + chunk * CH, 8), nrows),
                       pl.ds(c * HH, HH)])

    @pl.when(s < NS - 1)
    def _():
        stage_in(0, CH, 0)
        stage_in(1, CH, 1)
        stage_out(0, CH, 0)
        stage_in(2, CH, 0)
        stage_out(1, CH, 1)
        stage_in(3, CH, 1)
        stage_out(2, CH, 0)
        stage_in(4, CH, 0)
        stage_out(3, CH, 1)
        stage_out(4, CH, 0)

    @pl.when(s == NS - 1)
    def _():
        TAIL = N - (NS - 1) * RPT - 3 * CH
        stage_in(0, CH, 0)
        stage_in(1, CH, 1)
        stage_out(0, CH, 0)
        stage_in(2, CH, 0)
        stage_out(1, CH, 1)
        stage_in(3, TAIL, 1)
        stage_out(2, CH, 0)
        stage_out(3, TAIL, 1)


@functools.cache
def _sc_agg_kernel():
    return pl.kernel(
        _sc_agg_body,
        out_type=jax.ShapeDtypeStruct((N, H), jnp.float32),
        mesh=_mesh(),
        compiler_params=pltpu.CompilerParams(use_tc_tiling_on_sc=False),
        scratch_types=[
            pltpu.VMEM((UPT + 1, 128), jnp.int32),
            pltpu.VMEM((UPT + 1, 128), jnp.int32),
            pltpu.VMEM((128, HH), jnp.float32),
            pltpu.VMEM((128, HH), jnp.float32),
            pltpu.VMEM((128, HH), jnp.float32),
            pltpu.VMEM((128, HH), jnp.float32),
            pltpu.VMEM((CH, HH), jnp.float32),
            pltpu.VMEM_SHARED((NP_, HH), jnp.float32),
            pltpu.SemaphoreType.DMA((4,)),
            pltpu.SemaphoreType.DMA((4,)),
            pltpu.SemaphoreType.DMA,
        ],
    )


def _sc_agg(g, src2, dst2):
    return _sc_agg_kernel()(g.reshape(NC * N, HH), src2, dst2)


def _prep_body(x_ref, w_ref, dinv_ref, g_ref):
    u = jnp.dot(x_ref[...], w_ref[...], preferred_element_type=jnp.float32)
    g_ref[...] = u * dinv_ref[...]


def _tc_prep(x, W1, dinvb):
    return pl.pallas_call(
        _prep_body,
        grid=(NB,),
        in_specs=[
            pl.BlockSpec((BN, D), lambda i: (i, 0)),
            pl.BlockSpec((D, H), lambda i: (0, 0)),
            pl.BlockSpec((BN, H), lambda i: (i, 0)),
        ],
        out_specs=pl.BlockSpec((BN, H), lambda i: (i, 0)),
        out_shape=jax.ShapeDtypeStruct((N, H), jnp.float32),
    )(x, W1, dinvb)


def _mid_body(p0_ref, g_ref, dinv_ref, b_ref, w_ref, o_ref):
    z = dinv_ref[...] * (p0_ref[...] + g_ref[...]) + b_ref[...]
    z = jnp.maximum(z, 0.0)
    u = jnp.dot(z, w_ref[...], preferred_element_type=jnp.float32)
    o_ref[...] = u * dinv_ref[...]


def _tc_mid(agg, g, dinvb, b, W):
    return pl.pallas_call(
        _mid_body,
        grid=(NB,),
        in_specs=[
            pl.BlockSpec((BN, H), lambda i: (i, 0)),
            pl.BlockSpec((BN, H), lambda i: (i, 0)),
            pl.BlockSpec((BN, H), lambda i: (i, 0)),
            pl.BlockSpec((1, H), lambda i: (0, 0)),
            pl.BlockSpec((H, H), lambda i: (0, 0)),
        ],
        out_specs=pl.BlockSpec((BN, H), lambda i: (i, 0)),
        out_shape=jax.ShapeDtypeStruct((N, H), jnp.float32),
    )(agg, g, dinvb, b, W)


def _final_body(p0_ref, g_ref, dinv_ref, b_ref, batch_ref, wo_ref,
                bo_ref, o_ref, acc, cnt):
    i = pl.program_id(0)

    @pl.when(i == 0)
    def _():
        acc[...] = jnp.zeros_like(acc)
        cnt[...] = jnp.zeros_like(cnt)

    z = dinv_ref[...] * (p0_ref[...] + g_ref[...]) + b_ref[...]
    z = jnp.maximum(z, 0.0)
    q = jnp.dot(z, wo_ref[...], preferred_element_type=jnp.float32)
    gids = lax.broadcasted_iota(jnp.int32, (BN, G), 1)
    m = (batch_ref[...] == gids).astype(jnp.float32)
    dn = (((0,), (0,)), ((), ()))
    acc[...] += lax.dot_general(m, q, dn, preferred_element_type=jnp.float32)
    cnt[...] += lax.dot_general(m, jnp.ones((BN, 1), jnp.float32), dn,
                                preferred_element_type=jnp.float32)

    @pl.when(i == NB - 1)
    def _():
        o_ref[...] = acc[...] / jnp.maximum(cnt[...], 1.0) + bo_ref[...]


def _tc_final(agg, g, dinvb, b, batchb, Wo, bo2d):
    return pl.pallas_call(
        _final_body,
        grid=(NB,),
        in_specs=[
            pl.BlockSpec((BN, H), lambda i: (i, 0)),
            pl.BlockSpec((BN, H), lambda i: (i, 0)),
            pl.BlockSpec((BN, H), lambda i: (i, 0)),
            pl.BlockSpec((1, H), lambda i: (0, 0)),
            pl.BlockSpec((BN, G), lambda i: (i, 0)),
            pl.BlockSpec((H, 1), lambda i: (0, 0)),
            pl.BlockSpec((1, 1), lambda i: (0, 0)),
        ],
        out_specs=pl.BlockSpec((G, 1), lambda i: (0, 0)),
        out_shape=jax.ShapeDtypeStruct((G, 1), jnp.float32),
        scratch_shapes=[
            pltpu.VMEM((G, 1), jnp.float32),
            pltpu.VMEM((G, 1), jnp.float32),
        ],
    )(agg, g, dinvb, b, batchb, Wo, bo2d)


def kernel(x, edge_index, batch, W1, b1, W2, b2, Wo, bo):
    degp, src2, dst2 = _sc_deg(edge_index)
    dinv = lax.rsqrt(degp[:N] + degp[NDP:NDP + N] + 1.0)
    dinvb = jnp.broadcast_to(dinv[:, None], (N, H))
    batchb = jnp.broadcast_to(batch[:, None], (N, G))

    g1 = _tc_prep(x, W1, dinvb)
    a1 = _sc_agg(g1, src2, dst2)
    g2 = _tc_mid(a1, g1, dinvb, b1.reshape(1, H), W2)
    a2 = _sc_agg(g2, src2, dst2)
    out = _tc_final(a2, g2, dinvb, b2.reshape(1, H), batchb,
                    Wo, bo.reshape(1, 1))
    return out.reshape(-1)


# speedup vs baseline: 35.7057x; 1.0106x over previous
import functools

import jax
import jax.numpy as jnp
from jax import lax
from jax.experimental import pallas as pl
from jax.experimental.pallas import tpu as pltpu
from jax.experimental.pallas import tpu_sc as plsc

N = 10000
E = 320000
D = 128
H = 128
G = 128
HH = H // 2

NC = 2
NS = 16
NW = NC * NS

EU = E // 128
EUP = 2504
UPW = 80
UPA = 78
UPT = 156

NP_ = 10240
RPT = NP_ // NS
CH = RPT // 5
NDP = 10240

BN = 1000
NB = N // BN


def _mesh():
    return plsc.VectorSubcoreMesh(core_axis_name="c", subcore_axis_name="s",
                                  num_cores=NC, num_subcores=NS)


def _sc_deg_body(ei_hbm, deg_hbm, src_hbm, dst_hbm,
                 ubuf, ones_v, zb, deg_sh, sem, wsem):
    c = lax.axis_index("c")
    s = lax.axis_index("s")
    w = s * NC + c
    base = w * UPW

    one16 = jnp.ones((16,), jnp.float32)
    zero16 = jnp.zeros((16,), jnp.float32)
    for j in range(128 // 16):
        ones_v[pl.ds(j * 16, 16)] = one16

    @pl.loop(0, 1024 // 16)
    def _(i):
        zb[pl.ds(i * 16, 16)] = zero16

    @pl.when(s < 10)
    def _():
        pltpu.sync_copy(zb, deg_sh.at[pl.ds(pl.multiple_of(s * 1024, 8),
                                            1024)])

    AHEAD = 8
    plsc.subcore_barrier()

    def stage_and_hist(nu):
        @pl.loop(0, nu)
        def _(u):
            pltpu.async_copy(
                ei_hbm.at[:, pl.ds(pl.multiple_of(128 * (base + u), 128),
                                   128)],
                ubuf.at[u], sem)

            @pl.when(u >= AHEAD)
            def _():
                pltpu.make_async_copy(ei_hbm.at[:, pl.ds(0, 128)],
                                      ubuf.at[0], sem).wait()
                pltpu.async_copy(ones_v, deg_sh.at[ubuf.at[u - AHEAD, 1]],
                                 wsem, add=True)

            @pl.when(u >= 2 * AHEAD)
            def _():
                pltpu.make_async_copy(ones_v, deg_sh.at[ubuf.at[0, 1]],
                                      wsem).wait()

        for _ in range(AHEAD):
            pltpu.make_async_copy(ei_hbm.at[:, pl.ds(0, 128)], ubuf.at[0],
                                  sem).wait()
        for k in range(AHEAD):
            pltpu.async_copy(ones_v, deg_sh.at[ubuf.at[nu - AHEAD + k, 1]],
                             wsem, add=True)
        for _ in range(min(2 * AHEAD, nu)):
            pltpu.make_async_copy(ones_v, deg_sh.at[ubuf.at[0, 1]],
                                  wsem).wait()

    def write_units(nrows):
        pltpu.sync_copy(ubuf.at[pl.ds(0, nrows), 0],
                        src_hbm.at[pl.ds(pl.multiple_of(base, 8), nrows)])
        pltpu.sync_copy(ubuf.at[pl.ds(0, nrows), 1],
                        dst_hbm.at[pl.ds(pl.multiple_of(base, 8), nrows)])

    @pl.when(w < NW - 1)
    def _():
        stage_and_hist(UPW)
        write_units(UPW)

    @pl.when(w == NW - 1)
    def _():
        stage_and_hist(EU - (NW - 1) * UPW)
        write_units(EUP - (NW - 1) * UPW)

    plsc.subcore_barrier()

    @pl.when(s < 10)
    def _():
        pltpu.sync_copy(deg_sh.at[pl.ds(pl.multiple_of(s * 1024, 8), 1024)],
                        zb)
        pltpu.sync_copy(zb,
                        deg_hbm.at[pl.ds(pl.multiple_of(
                            c * NDP + s * 1024, 8), 1024)])


@functools.cache
def _sc_deg_kernel():
    return pl.kernel(
        _sc_deg_body,
        out_type=(jax.ShapeDtypeStruct((NC * NDP,), jnp.float32),
                  jax.ShapeDtypeStruct((EUP, 128), jnp.int32),
                  jax.ShapeDtypeStruct((EUP, 128), jnp.int32)),
        mesh=_mesh(),
        compiler_params=pltpu.CompilerParams(use_tc_tiling_on_sc=True),
        scratch_types=[
            pltpu.VMEM((UPW, 2, 128), jnp.int32),
            pltpu.VMEM((128,), jnp.float32),
            pltpu.VMEM((1024,), jnp.float32),
            pltpu.VMEM_SHARED((NDP,), jnp.float32),
            pltpu.SemaphoreType.DMA,
            pltpu.SemaphoreType.DMA,
        ],
    )


def _sc_deg(edge_index):
    return _sc_deg_kernel()(edge_index)


def _sc_agg_body(g_hbm, src_hbm, dst_hbm, out_hbm,
                 srcb, dstb, bufs0, bufs1, bufs2, bufs3, zc,
                 acc_sh, gsems, ssems, zsem):
    c = lax.axis_index("c")
    s = lax.axis_index("s")
    bufs = (bufs0, bufs1, bufs2, bufs3)

    zero16 = jnp.zeros((16,), jnp.float32)

    @pl.loop(0, CH)
    def _(r):
        for j in range(HH // 16):
            zc[r, pl.ds(j * 16, 16)] = zero16

    pltpu.sync_copy(src_hbm.at[pl.ds(s * UPT, UPT)], srcb.at[pl.ds(0, UPT)])
    pltpu.sync_copy(dst_hbm.at[pl.ds(s * UPT, UPT)], dstb.at[pl.ds(0, UPT)])

    @pl.when(s < 4)
    def _():
        pltpu.sync_copy(src_hbm.at[pl.ds(NS * UPT + s, 1)],
                        srcb.at[pl.ds(UPT, 1)])
        pltpu.sync_copy(dst_hbm.at[pl.ds(NS * UPT + s, 1)],
                        dstb.at[pl.ds(UPT, 1)])

    for t in range(5):
        pltpu.async_copy(
            zc, acc_sh.at[pl.ds(pl.multiple_of(s * RPT + t * CH, 8), CH)],
            zsem)
    for t in range(5):
        pltpu.make_async_copy(
            zc, acc_sh.at[pl.ds(pl.multiple_of(s * RPT + t * CH, 8), CH)],
            zsem).wait()

    @pl.loop(0, UPT + 1)
    def _(j):
        for q in range(128 // 16):
            srcb[j, pl.ds(q * 16, 16)] = srcb[j, pl.ds(q * 16, 16)] * 2 + c

    plsc.subcore_barrier()

    def g_start(j, l):
        pltpu.async_copy(g_hbm.at[srcb.at[j]], bufs[l], gsems.at[l])

    def g_wait(j, l):
        pltpu.make_async_copy(g_hbm.at[srcb.at[j]], bufs[l],
                              gsems.at[l]).wait()

    def s_start(j, l):
        pltpu.async_copy(bufs[l], acc_sh.at[dstb.at[j]], ssems.at[l],
                         add=True)

    def s_wait(j, l):
        pltpu.make_async_copy(bufs[l], acc_sh.at[dstb.at[j]],
                              ssems.at[l]).wait()

    for l in range(4):
        g_start(l, l)

    @pl.loop(0, (UPT - 4) // 4)
    def _(t):
        j0 = 4 * t
        for l in range(4):
            g_wait(j0 + l, l)
            s_start(j0 + l, l)
        for l in range(4):
            s_wait(j0 + l, l)
            g_start(j0 + l + 4, l)

    for l in range(4):
        g_wait(UPT - 4 + l, l)
        s_start(UPT - 4 + l, l)
    s_wait(UPT - 4, 0)

    @pl.when(s < 4)
    def _():
        g_start(UPT, 0)
        g_wait(UPT, 0)
        s_start(UPT, 0)
        s_wait(UPT, 0)

    for l in range(1, 4):
        s_wait(UPT - 4 + l, l)

    plsc.subcore_barrier()

    zbs = (bufs0, bufs1)

    def stage_in(chunk, nrows, l):
        base = pl.multiple_of(s * RPT + chunk * CH, 8)
        pltpu.async_copy(acc_sh.at[pl.ds(base, nrows)],
                         zbs[l].at[pl.ds(0, nrows)], gsems.at[l])

    def stage_out(chunk, nrows, l):
        base = pl.multiple_of(s * RPT + chunk * CH, 8)
        pltpu.make_async_copy(acc_sh.at[pl.ds(base, nrows)],
                              zbs[l].at[pl.ds(0, nrows)],
                              gsems.at[l]).wait()
        pltpu.sync_copy(
            zbs[l].at[pl.ds(0, nrows)],
            out_hbm.at[pl.ds(pl.multiple_of(s * RPT + chunk * CH, 8), nrows),
                       pl.ds(c * HH, HH)])

    @pl.when(s < NS - 1)
    def _():
        stage_in(0, CH, 0)
        stage_in(1, CH, 1)
        stage_out(0, CH, 0)
        stage_in(2, CH, 0)
        stage_out(1, CH, 1)
        stage_in(3, CH, 1)
        stage_out(2, CH, 0)
        stage_in(4, CH, 0)
        stage_out(3, CH, 1)
        stage_out(4, CH, 0)

    @pl.when(s == NS - 1)
    def _():
        TAIL = N - (NS - 1) * RPT - 3 * CH
        stage_in(0, CH, 0)
        stage_in(1, CH, 1)
        stage_out(0, CH, 0)
        stage_in(2, CH, 0)
        stage_out(1, CH, 1)
        stage_in(3, TAIL, 1)
        stage_out(2, CH, 0)
        stage_out(3, TAIL, 1)


@functools.cache
def _sc_agg_kernel():
    return pl.kernel(
        _sc_agg_body,
        out_type=jax.ShapeDtypeStruct((N, H), jnp.float32),
        mesh=_mesh(),
        compiler_params=pltpu.CompilerParams(use_tc_tiling_on_sc=False),
        scratch_types=[
            pltpu.VMEM((UPT + 1, 128), jnp.int32),
            pltpu.VMEM((UPT + 1, 128), jnp.int32),
            pltpu.VMEM((128, HH), jnp.float32),
            pltpu.VMEM((128, HH), jnp.float32),
            pltpu.VMEM((128, HH), jnp.float32),
            pltpu.VMEM((128, HH), jnp.float32),
            pltpu.VMEM((CH, HH), jnp.float32),
            pltpu.VMEM_SHARED((NP_, HH), jnp.float32),
            pltpu.SemaphoreType.DMA((4,)),
            pltpu.SemaphoreType.DMA((4,)),
            pltpu.SemaphoreType.DMA,
        ],
    )


def _sc_agg(g, src2, dst2):
    return _sc_agg_kernel()(g.reshape(NC * N, HH), src2, dst2)


def _prep_body(x_ref, w_ref, dinv_ref, g_ref):
    u = jnp.dot(x_ref[...], w_ref[...], preferred_element_type=jnp.float32)
    g_ref[...] = u * dinv_ref[...]


def _tc_prep(x, W1, dinvb):
    return pl.pallas_call(
        _prep_body,
        grid=(NB,),
        in_specs=[
            pl.BlockSpec((BN, D), lambda i: (i, 0)),
            pl.BlockSpec((D, H), lambda i: (0, 0)),
            pl.BlockSpec((BN, H), lambda i: (i, 0)),
        ],
        out_specs=pl.BlockSpec((BN, H), lambda i: (i, 0)),
        out_shape=jax.ShapeDtypeStruct((N, H), jnp.float32),
    )(x, W1, dinvb)


def _mid_body(p0_ref, g_ref, dinv_ref, b_ref, w_ref, o_ref):
    z = dinv_ref[...] * (p0_ref[...] + g_ref[...]) + b_ref[...]
    z = jnp.maximum(z, 0.0)
    u = jnp.dot(z, w_ref[...], preferred_element_type=jnp.float32)
    o_ref[...] = u * dinv_ref[...]


def _tc_mid(agg, g, dinvb, b, W):
    return pl.pallas_call(
        _mid_body,
        grid=(NB,),
        in_specs=[
            pl.BlockSpec((BN, H), lambda i: (i, 0)),
            pl.BlockSpec((BN, H), lambda i: (i, 0)),
            pl.BlockSpec((BN, H), lambda i: (i, 0)),
            pl.BlockSpec((1, H), lambda i: (0, 0)),
            pl.BlockSpec((H, H), lambda i: (0, 0)),
        ],
        out_specs=pl.BlockSpec((BN, H), lambda i: (i, 0)),
        out_shape=jax.ShapeDtypeStruct((N, H), jnp.float32),
    )(agg, g, dinvb, b, W)


def _final_body(p0_ref, g_ref, dinv_ref, b_ref, batch_ref, wo_ref,
                bo_ref, o_ref, acc, cnt):
    i = pl.program_id(0)

    @pl.when(i == 0)
    def _():
        acc[...] = jnp.zeros_like(acc)
        cnt[...] = jnp.zeros_like(cnt)

    z = dinv_ref[...] * (p0_ref[...] + g_ref[...]) + b_ref[...]
    z = jnp.maximum(z, 0.0)
    q = jnp.dot(z, wo_ref[...], preferred_element_type=jnp.float32)
    gids = lax.broadcasted_iota(jnp.int32, (BN, G), 1)
    m = (batch_ref[...].astype(jnp.int32) == gids).astype(jnp.float32)
    dn = (((0,), (0,)), ((), ()))
    acc[...] += lax.dot_general(m, q, dn, preferred_element_type=jnp.float32)
    cnt[...] += lax.dot_general(m, jnp.ones((BN, 1), jnp.float32), dn,
                                preferred_element_type=jnp.float32)

    @pl.when(i == NB - 1)
    def _():
        o_ref[...] = acc[...] / jnp.maximum(cnt[...], 1.0) + bo_ref[...]


def _tc_final(agg, g, dinvb, b, batchb, Wo, bo2d):
    return pl.pallas_call(
        _final_body,
        grid=(NB,),
        in_specs=[
            pl.BlockSpec((BN, H), lambda i: (i, 0)),
            pl.BlockSpec((BN, H), lambda i: (i, 0)),
            pl.BlockSpec((BN, H), lambda i: (i, 0)),
            pl.BlockSpec((1, H), lambda i: (0, 0)),
            pl.BlockSpec((BN, G), lambda i: (i, 0)),
            pl.BlockSpec((H, 1), lambda i: (0, 0)),
            pl.BlockSpec((1, 1), lambda i: (0, 0)),
        ],
        out_specs=pl.BlockSpec((G, 1), lambda i: (0, 0)),
        out_shape=jax.ShapeDtypeStruct((G, 1), jnp.float32),
        scratch_shapes=[
            pltpu.VMEM((G, 1), jnp.float32),
            pltpu.VMEM((G, 1), jnp.float32),
        ],
    )(agg, g, dinvb, b, batchb, Wo, bo2d)


def kernel(x, edge_index, batch, W1, b1, W2, b2, Wo, bo):
    degp, src2, dst2 = _sc_deg(edge_index)
    dinv = lax.rsqrt(degp[:N] + degp[NDP:NDP + N] + 1.0)
    dinvb = jnp.broadcast_to(dinv[:, None], (N, H))
    batchb = jnp.broadcast_to(batch.astype(jnp.int8)[:, None], (N, G))

    g1 = _tc_prep(x, W1, dinvb)
    a1 = _sc_agg(g1, src2, dst2)
    g2 = _tc_mid(a1, g1, dinvb, b1.reshape(1, H), W2)
    a2 = _sc_agg(g2, src2, dst2)
    out = _tc_final(a2, g2, dinvb, b2.reshape(1, H), batchb,
                    Wo, bo.reshape(1, 1))
    return out.reshape(-1)
